# baseline (speedup 1.0000x reference)
import functools

import jax
import jax.numpy as jnp
from jax import lax
from jax.experimental import pallas as pl
from jax.experimental.pallas import tpu as pltpu
from jax.experimental.pallas import tpu_sc as plsc

NC, NS = 2, 16
NW = NC * NS
B, L, D = 4096, 200, 32
CH = 128
LP = 2 * CH
BPW = B // NW
NCH = 2 * BPW
NBUF = 16
LANES = 16


def _build(interpret=False):
    mesh = plsc.VectorSubcoreMesh(
        core_axis_name="c", subcore_axis_name="s",
        num_cores=NC, num_subcores=NS)

    @functools.partial(
        pl.kernel,
        out_type=jax.ShapeDtypeStruct((NW, BPW, D), jnp.float32),
        mesh=mesh,
        scratch_types=(
            [pltpu.VMEM((2 * BPW, CH), jnp.int32)]
            + [pltpu.VMEM((CH, D), jnp.float32) for _ in range(NBUF)]
            + [pltpu.VMEM((BPW, D), jnp.float32)]
            + [pltpu.SemaphoreType.DMA for _ in range(NBUF)]
        ),
        compiler_params=pltpu.CompilerParams(use_tc_tiling_on_sc=False),
        interpret=interpret,
    )
    def set_embed(x_hbm, table_hbm, out_hbm, idx_v, *rest):
        bufs = rest[:NBUF]
        out_v = rest[NBUF]
        sems = rest[NBUF + 1:2 * NBUF + 1]
        wid = lax.axis_index("c") * NS + lax.axis_index("s")
        pltpu.sync_copy(x_hbm.at[wid], idx_v)

        def gather_start(c, rows, sem):
            pltpu.async_copy(table_hbm.at[idx_v.at[c]], rows, sem)

        def gather_wait(c, rows, sem):
            pltpu.make_async_copy(table_hbm.at[idx_v.at[c]], rows, sem).wait()

        def prefetch(c, rows, sem):
            @pl.when(c < NCH)
            def _pf():
                gather_start(c, rows, sem)

        def reduce_chunk(rows, carry_in):
            def body(t, carry):
                v0, v1, w0, w1 = carry
                for u in range(4):
                    j = t * 8 + 2 * u
                    v0 = jnp.maximum(v0, rows[j, pl.ds(0, LANES)])
                    v1 = jnp.maximum(v1, rows[j, pl.ds(LANES, LANES)])
                    w0 = jnp.maximum(w0, rows[j + 1, pl.ds(0, LANES)])
                    w1 = jnp.maximum(w1, rows[j + 1, pl.ds(LANES, LANES)])
                return v0, v1, w0, w1
            return lax.fori_loop(0, CH // 8, body, carry_in)

        for k in range(NBUF):
            gather_start(k, bufs[k], sems[k])
        neg = jnp.full((LANES,), -jnp.inf, jnp.float32)

        def outer(p, _):
            c0 = p * NBUF
            for k in range(0, NBUF, 2):
                ca, cb = c0 + k, c0 + k + 1
                gather_wait(ca, bufs[k], sems[k])
                v = reduce_chunk(bufs[k], (neg, neg, neg, neg))
                prefetch(ca + NBUF, bufs[k], sems[k])
                gather_wait(cb, bufs[k + 1], sems[k + 1])
                v0, v1, w0, w1 = reduce_chunk(bufs[k + 1], v)
                r = p * (NBUF // 2) + k // 2
                out_v[r, pl.ds(0, LANES)] = jnp.maximum(v0, w0)
                out_v[r, pl.ds(LANES, LANES)] = jnp.maximum(v1, w1)
                prefetch(cb + NBUF, bufs[k + 1], sems[k + 1])
            return 0

        lax.fori_loop(0, NCH // NBUF, outer, 0)
        pltpu.sync_copy(out_v, out_hbm.at[wid])

    return set_embed


_set_embed = _build()

_TW = 65536


_G = _TW // 4


def _tp_body(in_ref, out_ref):
    x = in_ref[...]
    slab = jnp.concatenate(
        [x[:, k * _G:(k + 1) * _G] for k in range(4)], axis=0)
    out_ref[...] = slab.T


def _tc_transpose(tt):
    n = tt.shape[1]
    nb = pl.cdiv(n, _TW)
    return pl.pallas_call(
        _tp_body,
        grid=(nb,),
        in_specs=[pl.BlockSpec((D, _TW), lambda i: (0, i))],
        out_specs=pl.BlockSpec((_G, 128), lambda i: (i, 0)),
        out_shape=jax.ShapeDtypeStruct((nb * _G, 128), jnp.float32),
    )(tt)


def kernel(x, table):
    t2 = _tc_transpose(jnp.swapaxes(table, 0, 1))
    table_rm = t2.reshape(t2.shape[0] * 4, D)
    xp = jnp.concatenate([x, x[:, : LP - L]], axis=1)
    gl = _G.bit_length() - 1
    xp = (xp & ~(_TW - 1)) + ((xp & (_G - 1)) << 2) + ((xp >> gl) & 3)
    out = _set_embed(xp.reshape(NW, 2 * BPW, CH), table_rm)
    return out.reshape(B, 1, D)


# speedup vs baseline: 1.0376x; 1.0376x over previous
import functools

import jax
import jax.numpy as jnp
from jax import lax
from jax.experimental import pallas as pl
from jax.experimental.pallas import tpu as pltpu
from jax.experimental.pallas import tpu_sc as plsc

NC, NS = 2, 16
NW = NC * NS
B, L, D = 4096, 200, 32
CH = 128
LP = 2 * CH
BPW = B // NW
NCH = 2 * BPW
NBUF = 8
LANES = 16


def _build(interpret=False):
    mesh = plsc.VectorSubcoreMesh(
        core_axis_name="c", subcore_axis_name="s",
        num_cores=NC, num_subcores=NS)

    @functools.partial(
        pl.kernel,
        out_type=jax.ShapeDtypeStruct((NW, BPW, D), jnp.float32),
        mesh=mesh,
        scratch_types=(
            [pltpu.VMEM((2 * BPW, CH), jnp.int32)]
            + [pltpu.VMEM((CH, D), jnp.float32) for _ in range(NBUF)]
            + [pltpu.VMEM((BPW, D), jnp.float32)]
            + [pltpu.SemaphoreType.DMA for _ in range(NBUF)]
        ),
        compiler_params=pltpu.CompilerParams(use_tc_tiling_on_sc=False),
        interpret=interpret,
    )
    def set_embed(x_hbm, table_hbm, out_hbm, idx_v, *rest):
        bufs = rest[:NBUF]
        out_v = rest[NBUF]
        sems = rest[NBUF + 1:2 * NBUF + 1]
        wid = lax.axis_index("c") * NS + lax.axis_index("s")
        pltpu.sync_copy(x_hbm.at[wid], idx_v)

        def gather_start(c, rows, sem):
            pltpu.async_copy(table_hbm.at[idx_v.at[c]], rows, sem)

        def gather_wait(c, rows, sem):
            pltpu.make_async_copy(table_hbm.at[idx_v.at[c]], rows, sem).wait()

        def prefetch(c, rows, sem):
            @pl.when(c < NCH)
            def _pf():
                gather_start(c, rows, sem)

        def reduce_chunk(rows, carry_in):
            def body(t, carry):
                v0, v1, w0, w1 = carry
                for u in range(4):
                    j = t * 8 + 2 * u
                    v0 = jnp.maximum(v0, rows[j, pl.ds(0, LANES)])
                    v1 = jnp.maximum(v1, rows[j, pl.ds(LANES, LANES)])
                    w0 = jnp.maximum(w0, rows[j + 1, pl.ds(0, LANES)])
                    w1 = jnp.maximum(w1, rows[j + 1, pl.ds(LANES, LANES)])
                return v0, v1, w0, w1
            return lax.fori_loop(0, CH // 8, body, carry_in)

        for k in range(NBUF):
            gather_start(k, bufs[k], sems[k])
        neg = jnp.full((LANES,), -jnp.inf, jnp.float32)

        def outer(p, _):
            c0 = p * NBUF
            for k in range(0, NBUF, 2):
                ca, cb = c0 + k, c0 + k + 1
                gather_wait(ca, bufs[k], sems[k])
                v = reduce_chunk(bufs[k], (neg, neg, neg, neg))
                prefetch(ca + NBUF, bufs[k], sems[k])
                gather_wait(cb, bufs[k + 1], sems[k + 1])
                v0, v1, w0, w1 = reduce_chunk(bufs[k + 1], v)
                r = p * (NBUF // 2) + k // 2
                out_v[r, pl.ds(0, LANES)] = jnp.maximum(v0, w0)
                out_v[r, pl.ds(LANES, LANES)] = jnp.maximum(v1, w1)
                prefetch(cb + NBUF, bufs[k + 1], sems[k + 1])
            return 0

        lax.fori_loop(0, NCH // NBUF, outer, 0)
        pltpu.sync_copy(out_v, out_hbm.at[wid])

    return set_embed


_set_embed = _build()

_TW = 65536


_G = _TW // 4


def _tp_body(in_ref, out_ref):
    x = in_ref[...]
    slab = jnp.concatenate(
        [x[:, k * _G:(k + 1) * _G] for k in range(4)], axis=0)
    out_ref[...] = slab.T


def _tc_transpose(tt):
    n = tt.shape[1]
    nb = pl.cdiv(n, _TW)
    return pl.pallas_call(
        _tp_body,
        grid=(nb,),
        in_specs=[pl.BlockSpec((D, _TW), lambda i: (0, i))],
        out_specs=pl.BlockSpec((_G, 128), lambda i: (i, 0)),
        out_shape=jax.ShapeDtypeStruct((nb * _G, 128), jnp.float32),
    )(tt)


def kernel(x, table):
    t2 = _tc_transpose(jnp.swapaxes(table, 0, 1))
    table_rm = t2.reshape(t2.shape[0] * 4, D)
    xp = jnp.concatenate([x, x[:, : LP - L]], axis=1)
    gl = _G.bit_length() - 1
    xp = (xp & ~(_TW - 1)) + ((xp & (_G - 1)) << 2) + ((xp >> gl) & 3)
    out = _set_embed(xp.reshape(NW, 2 * BPW, CH), table_rm)
    return out.reshape(B, 1, D)


# speedup vs baseline: 1.2537x; 1.2083x over previous
import functools

import jax
import jax.numpy as jnp
from jax import lax
from jax.experimental import pallas as pl
from jax.experimental.pallas import tpu as pltpu
from jax.experimental.pallas import tpu_sc as plsc

NC, NS = 2, 16
NW = NC * NS
B, L, D = 4096, 200, 32
CH = 128
LP = 2 * CH
BPW = B // NW
NCH = 2 * BPW
NBUF = 8
DW = D // 2


def _build(interpret=False):
    mesh = plsc.VectorSubcoreMesh(
        core_axis_name="c", subcore_axis_name="s",
        num_cores=NC, num_subcores=NS)

    @functools.partial(
        pl.kernel,
        out_type=jax.ShapeDtypeStruct((NW, BPW, D), jnp.bfloat16),
        mesh=mesh,
        scratch_types=(
            [pltpu.VMEM((2 * BPW, CH), jnp.int32)]
            + [pltpu.VMEM((CH, DW), jnp.uint32) for _ in range(NBUF)]
            + [pltpu.VMEM((BPW, D), jnp.bfloat16)]
            + [pltpu.SemaphoreType.DMA for _ in range(NBUF)]
        ),
        compiler_params=pltpu.CompilerParams(
            use_tc_tiling_on_sc=False, needs_layout_passes=False),
        interpret=interpret,
    )
    def set_embed(x_hbm, table_hbm, out_hbm, idx_v, *rest):
        bufs = rest[:NBUF]
        out_v = rest[NBUF]
        sems = rest[NBUF + 1:2 * NBUF + 1]
        wid = lax.axis_index("c") * NS + lax.axis_index("s")
        pltpu.sync_copy(x_hbm.at[wid], idx_v)

        def gather_start(c, rows, sem):
            pltpu.async_copy(table_hbm.at[idx_v.at[c]], rows, sem)

        def gather_wait(c, rows, sem):
            pltpu.make_async_copy(table_hbm.at[idx_v.at[c]], rows, sem).wait()

        def prefetch(c, rows, sem):
            @pl.when(c < NCH)
            def _pf():
                gather_start(c, rows, sem)

        def ldrow(rows, j):
            return plsc.bitcast(rows[j, :], jnp.bfloat16)

        def reduce_chunk(rows, carry_in):
            def body(t, carry):
                a, b, c, d = carry
                j = t * 8
                a = jnp.maximum(a, ldrow(rows, j))
                b = jnp.maximum(b, ldrow(rows, j + 1))
                c = jnp.maximum(c, ldrow(rows, j + 2))
                d = jnp.maximum(d, ldrow(rows, j + 3))
                a = jnp.maximum(a, ldrow(rows, j + 4))
                b = jnp.maximum(b, ldrow(rows, j + 5))
                c = jnp.maximum(c, ldrow(rows, j + 6))
                d = jnp.maximum(d, ldrow(rows, j + 7))
                return a, b, c, d
            return lax.fori_loop(0, CH // 8, body, carry_in)

        for k in range(NBUF):
            gather_start(k, bufs[k], sems[k])
        neg = jnp.full((D,), -jnp.inf, jnp.bfloat16)

        def outer(p, _):
            c0 = p * NBUF
            for k in range(0, NBUF, 2):
                ca, cb = c0 + k, c0 + k + 1
                gather_wait(ca, bufs[k], sems[k])
                v = reduce_chunk(bufs[k], (neg, neg, neg, neg))
                prefetch(ca + NBUF, bufs[k], sems[k])
                gather_wait(cb, bufs[k + 1], sems[k + 1])
                a, b, c, d = reduce_chunk(bufs[k + 1], v)
                r = p * (NBUF // 2) + k // 2
                out_v[r, :] = jnp.maximum(jnp.maximum(a, b),
                                          jnp.maximum(c, d))
                prefetch(cb + NBUF, bufs[k + 1], sems[k + 1])
            return 0

        lax.fori_loop(0, NCH // NBUF, outer, 0)
        pltpu.sync_copy(out_v, out_hbm.at[wid])

    return set_embed


_set_embed = _build()

_TW = 65536
_G = _TW // 8


def _tp_body(in_ref, out_ref):
    x = in_ref[...]
    lo = lax.bitcast_convert_type(
        x[0:16, :].astype(jnp.bfloat16), jnp.uint16).astype(jnp.uint32)
    hi = lax.bitcast_convert_type(
        x[16:32, :].astype(jnp.bfloat16), jnp.uint16).astype(jnp.uint32)
    u = lo | (hi << 16)
    slab = jnp.concatenate(
        [u[:, k * _G:(k + 1) * _G] for k in range(8)], axis=0)
    out_ref[...] = slab.T


def _tc_transpose(tt):
    n = tt.shape[1]
    nb = pl.cdiv(n, _TW)
    return pl.pallas_call(
        _tp_body,
        grid=(nb,),
        in_specs=[pl.BlockSpec((D, _TW), lambda i: (0, i))],
        out_specs=pl.BlockSpec((_G, 128), lambda i: (i, 0)),
        out_shape=jax.ShapeDtypeStruct((nb * _G, 128), jnp.uint32),
    )(tt)


def kernel(x, table):
    t2 = _tc_transpose(jnp.swapaxes(table, 0, 1))
    table_pk = t2.reshape(t2.shape[0] * 8, DW)
    xp = jnp.concatenate([x, x[:, : LP - L]], axis=1)
    gl = _G.bit_length() - 1
    xp = (xp & ~(_TW - 1)) + ((xp & (_G - 1)) << 3) + ((xp >> gl) & 7)
    out = _set_embed(xp.reshape(NW, 2 * BPW, CH), table_pk)
    o = out.astype(jnp.float32).reshape(B, DW, 2)
    o = jnp.swapaxes(o, 1, 2).reshape(B, 1, D)
    return o


# speedup vs baseline: 1.2629x; 1.0073x over previous
import functools

import jax
import jax.numpy as jnp
from jax import lax
from jax.experimental import pallas as pl
from jax.experimental.pallas import tpu as pltpu
from jax.experimental.pallas import tpu_sc as plsc

NC, NS = 2, 16
NW = NC * NS
B, L, D = 4096, 200, 32
CH = 128
LP = 2 * CH
BPW = B // NW
NCH = 2 * BPW
NBUF = 8
DW = D // 2


def _build(interpret=False):
    mesh = plsc.VectorSubcoreMesh(
        core_axis_name="c", subcore_axis_name="s",
        num_cores=NC, num_subcores=NS)

    @functools.partial(
        pl.kernel,
        out_type=jax.ShapeDtypeStruct((NW, BPW, D), jnp.bfloat16),
        mesh=mesh,
        scratch_types=(
            [pltpu.VMEM((2 * BPW, CH), jnp.int32)]
            + [pltpu.VMEM((CH, DW), jnp.uint32) for _ in range(NBUF)]
            + [pltpu.VMEM((BPW, D), jnp.bfloat16)]
            + [pltpu.SemaphoreType.DMA for _ in range(NBUF)]
        ),
        compiler_params=pltpu.CompilerParams(
            use_tc_tiling_on_sc=False, needs_layout_passes=False),
        interpret=interpret,
    )
    def set_embed(x_hbm, table_hbm, out_hbm, idx_v, *rest):
        bufs = rest[:NBUF]
        out_v = rest[NBUF]
        sems = rest[NBUF + 1:2 * NBUF + 1]
        wid = lax.axis_index("c") * NS + lax.axis_index("s")
        pltpu.sync_copy(x_hbm.at[wid], idx_v)

        def gather_start(c, rows, sem):
            pltpu.async_copy(table_hbm.at[idx_v.at[c]], rows, sem)

        def gather_wait(c, rows, sem):
            pltpu.make_async_copy(table_hbm.at[idx_v.at[c]], rows, sem).wait()

        def prefetch(c, rows, sem):
            @pl.when(c < NCH)
            def _pf():
                gather_start(c, rows, sem)

        def ldrow(rows, j):
            return plsc.bitcast(rows[j, :], jnp.bfloat16)

        def reduce_chunk(rows, carry_in):
            def body(t, carry):
                a, b, c, d = carry
                j = t * 8
                a = jnp.maximum(a, ldrow(rows, j))
                b = jnp.maximum(b, ldrow(rows, j + 1))
                c = jnp.maximum(c, ldrow(rows, j + 2))
                d = jnp.maximum(d, ldrow(rows, j + 3))
                a = jnp.maximum(a, ldrow(rows, j + 4))
                b = jnp.maximum(b, ldrow(rows, j + 5))
                c = jnp.maximum(c, ldrow(rows, j + 6))
                d = jnp.maximum(d, ldrow(rows, j + 7))
                return a, b, c, d
            return lax.fori_loop(0, CH // 8, body, carry_in)

        for k in range(NBUF):
            gather_start(k, bufs[k], sems[k])
        neg = jnp.full((D,), -jnp.inf, jnp.bfloat16)

        def outer(p, _):
            c0 = p * NBUF
            for k in range(0, NBUF, 2):
                ca, cb = c0 + k, c0 + k + 1
                gather_wait(ca, bufs[k], sems[k])
                v = reduce_chunk(bufs[k], (neg, neg, neg, neg))
                prefetch(ca + NBUF, bufs[k], sems[k])
                gather_wait(cb, bufs[k + 1], sems[k + 1])
                a, b, c, d = reduce_chunk(bufs[k + 1], v)
                r = p * (NBUF // 2) + k // 2
                out_v[r, :] = jnp.maximum(jnp.maximum(a, b),
                                          jnp.maximum(c, d))
                prefetch(cb + NBUF, bufs[k + 1], sems[k + 1])
            return 0

        lax.fori_loop(0, NCH // NBUF, outer, 0)
        pltpu.sync_copy(out_v, out_hbm.at[wid])

    return set_embed


_set_embed = _build()

_TW = 131072
_G = _TW // 8


def _tp_body(in_ref, out_ref):
    x = in_ref[...]
    lo = lax.bitcast_convert_type(
        x[0:16, :].astype(jnp.bfloat16), jnp.uint16).astype(jnp.uint32)
    hi = lax.bitcast_convert_type(
        x[16:32, :].astype(jnp.bfloat16), jnp.uint16).astype(jnp.uint32)
    u = lo | (hi << 16)
    slab = jnp.concatenate(
        [u[:, k * _G:(k + 1) * _G] for k in range(8)], axis=0)
    out_ref[...] = slab.T


def _tc_transpose(tt):
    n = tt.shape[1]
    nb = pl.cdiv(n, _TW)
    return pl.pallas_call(
        _tp_body,
        grid=(nb,),
        in_specs=[pl.BlockSpec((D, _TW), lambda i: (0, i))],
        out_specs=pl.BlockSpec((_G, 128), lambda i: (i, 0)),
        out_shape=jax.ShapeDtypeStruct((nb * _G, 128), jnp.uint32),
    )(tt)


def kernel(x, table):
    t2 = _tc_transpose(jnp.swapaxes(table, 0, 1))
    table_pk = t2.reshape(t2.shape[0] * 8, DW)
    xp = jnp.concatenate([x, x[:, : LP - L]], axis=1)
    gl = _G.bit_length() - 1
    xp = (xp & ~(_TW - 1)) + ((xp & (_G - 1)) << 3) + ((xp >> gl) & 7)
    out = _set_embed(xp.reshape(NW, 2 * BPW, CH), table_pk)
    o = out.astype(jnp.float32).reshape(B, DW, 2)
    o = jnp.swapaxes(o, 1, 2).reshape(B, 1, D)
    return o
